# baseline (speedup 1.0000x reference)
import functools

import jax
import jax.numpy as jnp
from jax import lax
from jax.experimental import pallas as pl
from jax.experimental.pallas import tpu as pltpu
from jax.experimental.pallas import tpu_sc as plsc

N = 10000
E = 160000
D = 256
C = 64

NC = 2
NS = 16
L = 16
DH = D // NC

N_PAD = 10240
K = 128
E_PAD = 163840
CH_A = E_PAD // (NS * K)
ZROWS = N_PAD // NS


def _mesh():
    return plsc.VectorSubcoreMesh(core_axis_name="c", subcore_axis_name="s",
                                  num_cores=NC, num_subcores=NS)


NBUF = 2
ZB = 32


@functools.lru_cache(maxsize=None)
def _sc_agg(with_deg):
    scratch = (
        [pltpu.VMEM((K, DH), jnp.float32) for _ in range(NBUF)] +
        [pltpu.VMEM((2, K), jnp.int32) for _ in range(NBUF)] +
        [
            pltpu.VMEM((ZB, DH), jnp.float32),
            pltpu.VMEM((K,), jnp.float32),
            pltpu.VMEM((ZROWS,), jnp.float32),
            pltpu.VMEM_SHARED((N_PAD, DH), jnp.float32),
            pltpu.VMEM_SHARED((N_PAD,), jnp.float32),
        ] +
        [pltpu.SemaphoreType.DMA for _ in range(NBUF)]
    )

    def body(y_hbm, idx_hbm, out_hbm, deg_hbm, *sc):
        rows = list(sc[:NBUF])
        idx = list(sc[NBUF:2 * NBUF])
        zrow_v, ones_v, zdeg_v, acc_s, dacc_s = sc[2 * NBUF:2 * NBUF + 5]
        sems = list(sc[2 * NBUF + 5:])
        c = lax.axis_index("c")
        t = lax.axis_index("s")
        zeros16 = jnp.zeros((L,), jnp.float32)
        ones16 = jnp.ones((L,), jnp.float32)

        for i in range(ZB):
            for j in range(DH // L):
                zrow_v[i, pl.ds(j * L, L)] = zeros16
        for i in range(K // L):
            ones_v[pl.ds(i * L, L)] = ones16
        for i in range(ZROWS // L):
            zdeg_v[pl.ds(i * L, L)] = zeros16

        zbase = t * ZROWS
        for i in range(ZROWS // ZB):
            pltpu.async_copy(zrow_v, acc_s.at[pl.ds(zbase + i * ZB, ZB), :],
                             sems[0])
        pltpu.sync_copy(zdeg_v, dacc_s.at[pl.ds(zbase, ZROWS)])
        for i in range(ZROWS // ZB):
            pltpu.make_async_copy(zrow_v, acc_s.at[pl.ds(zbase + i * ZB, ZB), :],
                                  sems[0]).wait()

        rbase0 = 2 * (c * (E_PAD // K) + t * CH_A)

        def fire(ci, b):
            pltpu.sync_copy(idx_hbm.at[pl.ds(rbase0 + 2 * ci, 2)], idx[b])
            pltpu.async_copy(y_hbm.at[idx[b].at[0]], rows[b], sems[b])

        for b in range(NBUF - 1):
            fire(b, b)
        plsc.subcore_barrier()

        def group_body(g, _):
            for b in range(NBUF):
                ci = g * NBUF + b
                cf = ci + NBUF - 1

                @pl.when(cf < CH_A)
                def _():
                    fire(cf, (b + NBUF - 1) % NBUF)

                pltpu.make_async_copy(y_hbm.at[idx[b].at[0]],
                                      rows[b], sems[b]).wait()
                pltpu.sync_copy(rows[b], acc_s.at[idx[b].at[1]], add=True)

                if with_deg:
                    @pl.when(lax.rem(ci, 2) == c)
                    def _():
                        pltpu.sync_copy(ones_v, dacc_s.at[idx[b].at[1]],
                                        add=True)
            return ()

        lax.fori_loop(0, CH_A // NBUF, group_body, (), unroll=False)
        plsc.subcore_barrier()

        pltpu.sync_copy(acc_s.at[pl.ds(zbase, ZROWS)],
                        out_hbm.at[pl.ds(c * N_PAD + zbase, ZROWS)])

        if with_deg:
            pltpu.sync_copy(dacc_s.at[pl.ds(zbase, ZROWS)],
                            deg_hbm.at[pl.ds(c * N_PAD + zbase, ZROWS)])

    return pl.kernel(
        body,
        out_type=[jax.ShapeDtypeStruct((NC * N_PAD, DH), jnp.float32),
                  jax.ShapeDtypeStruct((NC * N_PAD,), jnp.float32)],
        mesh=_mesh(), scratch_types=scratch)


RB = 1024
GRID = N_PAD // RB

_f32 = jnp.float32


def _bdot(a, b):
    return jnp.dot(a.astype(jnp.bfloat16), b.astype(jnp.bfloat16),
                   preferred_element_type=_f32)


def _rowspec(w):
    return pl.BlockSpec((RB, w), lambda i: (i, 0))


def _stkspec():
    return pl.BlockSpec((NC, RB, DH), lambda i: (0, i, 0))


def _fullspec(shape):
    return pl.BlockSpec(shape, lambda i: tuple(0 for _ in shape))


def _split(y):
    return jnp.stack([y[:, :DH], y[:, DH:]], axis=0)


def _degspec():
    return pl.BlockSpec((NC, RB, 1), lambda i: (0, i, 0))


def _deg_of(deg_ref):
    return jnp.maximum(deg_ref[0] + deg_ref[1], 1.0)


def _tc_a1(feat, Wn0):
    def body(f_ref, wn_ref, y0_ref):
        y0_ref[...] = _split(_bdot(f_ref[...], wn_ref[...]))

    return pl.pallas_call(
        body,
        grid=(GRID,),
        in_specs=[_rowspec(D), _fullspec((D, D))],
        out_specs=_stkspec(),
        out_shape=jax.ShapeDtypeStruct((NC, N_PAD, DH), _f32),
    )(feat, Wn0)


def _tc_a2(feat, gfeat, Ws0, b0, Wpb, bp):
    def body(f_ref, g_ref, ws_ref, b0_ref, wpb_ref, bp_ref, s0_ref, gp_ref):
        s0_ref[...] = _bdot(f_ref[...], ws_ref[...]) + b0_ref[...]
        gp_ref[...] = _bdot(g_ref[...], wpb_ref[...]) + bp_ref[...]

    return pl.pallas_call(
        body,
        grid=(GRID,),
        in_specs=[_rowspec(D), _rowspec(D), _fullspec((D, D)),
                  _fullspec((1, D)), _fullspec((D, C)), _fullspec((1, C))],
        out_specs=[_rowspec(D), _rowspec(C)],
        out_shape=[jax.ShapeDtypeStruct((N_PAD, D), _f32),
                   jax.ShapeDtypeStruct((N_PAD, C), _f32)],
    )(feat, gfeat, Ws0, b0, Wpb, bp)


def _h1_of(sum_ref, deg_ref, s0_ref):
    d = _deg_of(deg_ref)
    agg = jnp.concatenate([sum_ref[0], sum_ref[1]], axis=-1).astype(_f32)
    return jnp.maximum(s0_ref[...] + agg / d, 0.0)


def _tc_b1(sum0, deg, s0, Wn1):
    def body(sum_ref, deg_ref, s0_ref, wn_ref, y1_ref):
        h1 = _h1_of(sum_ref, deg_ref, s0_ref)
        y1_ref[...] = _split(_bdot(h1, wn_ref[...]))

    return pl.pallas_call(
        body,
        grid=(GRID,),
        in_specs=[_stkspec(), _degspec(), _rowspec(D), _fullspec((D, D))],
        out_specs=_stkspec(),
        out_shape=jax.ShapeDtypeStruct((NC, N_PAD, DH), _f32),
    )(sum0, deg, s0, Wn1)


def _tc_b2(sum0, deg, s0, Ws1, b1):
    def body(sum_ref, deg_ref, s0_ref, ws_ref, b1_ref, s1_ref):
        h1 = _h1_of(sum_ref, deg_ref, s0_ref)
        s1_ref[...] = _bdot(h1, ws_ref[...]) + b1_ref[...]

    return pl.pallas_call(
        body,
        grid=(GRID,),
        in_specs=[_stkspec(), _degspec(), _rowspec(D), _fullspec((D, D)),
                  _fullspec((1, D))],
        out_specs=_rowspec(D),
        out_shape=jax.ShapeDtypeStruct((N_PAD, D), _f32),
    )(sum0, deg, s0, Ws1, b1)


def _tc_c(sum1, deg, s1, gp, Wpt):
    def body(sum_ref, deg_ref, s1_ref, gp_ref, wpt_ref, out_ref):
        h2 = _h1_of(sum_ref, deg_ref, s1_ref)
        out_ref[...] = _bdot(h2, wpt_ref[...]) + gp_ref[...]

    return pl.pallas_call(
        body,
        grid=(GRID,),
        in_specs=[_stkspec(), _degspec(), _rowspec(D), _rowspec(C),
                  _fullspec((D, C))],
        out_specs=_rowspec(C),
        out_shape=jax.ShapeDtypeStruct((N_PAD, C), _f32),
    )(sum1, deg, s1, gp, Wpt)


def kernel(feat, gfeat, edge_index, W_self0, W_neigh0, b0,
           W_self1, W_neigh1, b1, W_p, b_p):
    feat = jnp.pad(feat, ((0, N_PAD - N), (0, 0)))
    gfeat = jnp.pad(gfeat, ((0, N_PAD - N), (0, 0)))
    src = edge_index[0].astype(jnp.int32)
    dst = edge_index[1].astype(jnp.int32)
    pad = jnp.arange(E_PAD - E, dtype=jnp.int32)
    src = jnp.concatenate([src, pad % N])
    dst = jnp.concatenate([dst, N + pad % (N_PAD - N)])
    srcr = src.reshape(E_PAD // K, K)
    dstr = dst.reshape(E_PAD // K, K)
    idx_pack = jnp.concatenate([
        jnp.stack([srcr, dstr], axis=1),
        jnp.stack([srcr + N_PAD, dstr], axis=1),
    ]).reshape(NC * (E_PAD // K) * 2, K)

    b0r = b0.reshape(1, D)
    b1r = b1.reshape(1, D)
    bpr = b_p.reshape(1, C)
    Wpt = W_p[:D]
    Wpb = W_p[D:]

    y0 = _tc_a1(feat, W_neigh0)
    sum0, deg = _sc_agg(True)(y0.reshape(NC * N_PAD, DH), idx_pack)
    s0, gp = _tc_a2(feat, gfeat, W_self0, b0r, Wpb, bpr)
    degc = deg.reshape(NC, N_PAD, 1)
    sum0s = sum0.reshape(NC, N_PAD, DH)
    y1 = _tc_b1(sum0s, degc, s0, W_neigh1)
    sum1, _ = _sc_agg(False)(y1.reshape(NC * N_PAD, DH), idx_pack)
    s1 = _tc_b2(sum0s, degc, s0, W_self1, b1r)
    out = _tc_c(sum1.reshape(NC, N_PAD, DH), degc, s1, gp, Wpt)
    return out[:N]


# speedup vs baseline: 1.0014x; 1.0014x over previous
import functools

import jax
import jax.numpy as jnp
from jax import lax
from jax.experimental import pallas as pl
from jax.experimental.pallas import tpu as pltpu
from jax.experimental.pallas import tpu_sc as plsc

N = 10000
E = 160000
D = 256
C = 64

NC = 2
NS = 16
L = 16
DH = D // NC

N_PAD = 10240
K = 128
E_PAD = 163840
CH_A = E_PAD // (NS * K)
ZROWS = N_PAD // NS


def _mesh():
    return plsc.VectorSubcoreMesh(core_axis_name="c", subcore_axis_name="s",
                                  num_cores=NC, num_subcores=NS)


NBUF = 2
ZB = 32


@functools.lru_cache(maxsize=None)
def _sc_agg(with_deg):
    scratch = (
        [pltpu.VMEM((K, DH), jnp.float32) for _ in range(NBUF)] +
        [pltpu.VMEM((2, K), jnp.int32) for _ in range(NBUF)] +
        [
            pltpu.VMEM((ZB, DH), jnp.float32),
            pltpu.VMEM((K,), jnp.float32),
            pltpu.VMEM((ZROWS,), jnp.float32),
            pltpu.VMEM_SHARED((N_PAD, DH), jnp.float32),
            pltpu.VMEM_SHARED((N_PAD,), jnp.float32),
        ] +
        [pltpu.SemaphoreType.DMA for _ in range(NBUF)]
    )

    def body(y_hbm, idx_hbm, out_hbm, deg_hbm, *sc):
        rows = list(sc[:NBUF])
        idx = list(sc[NBUF:2 * NBUF])
        zrow_v, ones_v, zdeg_v, acc_s, dacc_s = sc[2 * NBUF:2 * NBUF + 5]
        sems = list(sc[2 * NBUF + 5:])
        c = lax.axis_index("c")
        t = lax.axis_index("s")
        zeros16 = jnp.zeros((L,), jnp.float32)
        ones16 = jnp.ones((L,), jnp.float32)

        for i in range(ZB):
            for j in range(DH // L):
                zrow_v[i, pl.ds(j * L, L)] = zeros16
        for i in range(K // L):
            ones_v[pl.ds(i * L, L)] = ones16
        for i in range(ZROWS // L):
            zdeg_v[pl.ds(i * L, L)] = zeros16

        zbase = t * ZROWS
        for i in range(ZROWS // ZB):
            pltpu.async_copy(zrow_v, acc_s.at[pl.ds(zbase + i * ZB, ZB), :],
                             sems[0])
        pltpu.sync_copy(zdeg_v, dacc_s.at[pl.ds(zbase, ZROWS)])
        for i in range(ZROWS // ZB):
            pltpu.make_async_copy(zrow_v, acc_s.at[pl.ds(zbase + i * ZB, ZB), :],
                                  sems[0]).wait()

        rbase0 = 2 * (c * (E_PAD // K) + t * CH_A)

        def fire(ci, b):
            pltpu.sync_copy(idx_hbm.at[pl.ds(rbase0 + 2 * ci, 2)], idx[b])
            pltpu.async_copy(y_hbm.at[idx[b].at[0]], rows[b], sems[b])

        for b in range(NBUF - 1):
            fire(b, b)
        plsc.subcore_barrier()

        def group_body(g, _):
            for b in range(NBUF):
                ci = g * NBUF + b
                cf = ci + NBUF - 1

                @pl.when(cf < CH_A)
                def _():
                    fire(cf, (b + NBUF - 1) % NBUF)

                pltpu.make_async_copy(y_hbm.at[idx[b].at[0]],
                                      rows[b], sems[b]).wait()
                pltpu.sync_copy(rows[b], acc_s.at[idx[b].at[1]], add=True)

                if with_deg:
                    @pl.when(lax.rem(ci, 2) == c)
                    def _():
                        pltpu.sync_copy(ones_v, dacc_s.at[idx[b].at[1]],
                                        add=True)
            return ()

        lax.fori_loop(0, CH_A // NBUF, group_body, (), unroll=False)
        plsc.subcore_barrier()

        pltpu.sync_copy(acc_s.at[pl.ds(zbase, ZROWS)],
                        out_hbm.at[pl.ds(c * N_PAD + zbase, ZROWS)])

        if with_deg:
            pltpu.sync_copy(dacc_s.at[pl.ds(zbase, ZROWS)],
                            deg_hbm.at[pl.ds(c * N_PAD + zbase, ZROWS)])

    return pl.kernel(
        body,
        out_type=[jax.ShapeDtypeStruct((NC * N_PAD, DH), jnp.float32),
                  jax.ShapeDtypeStruct((NC * N_PAD,), jnp.float32)],
        mesh=_mesh(), scratch_types=scratch)


RB = 1024
GRID = N_PAD // RB

_f32 = jnp.float32


def _bdot(a, b):
    return jnp.dot(a, b, preferred_element_type=_f32)


def _rowspec(w):
    return pl.BlockSpec((RB, w), lambda i: (i, 0))


def _stkspec():
    return pl.BlockSpec((NC, RB, DH), lambda i: (0, i, 0))


def _fullspec(shape):
    return pl.BlockSpec(shape, lambda i: tuple(0 for _ in shape))


def _split(y):
    return jnp.stack([y[:, :DH], y[:, DH:]], axis=0)


def _degspec():
    return pl.BlockSpec((NC, RB, 1), lambda i: (0, i, 0))


def _deg_of(deg_ref):
    return jnp.maximum(deg_ref[0] + deg_ref[1], 1.0)


def _tc_a1(feat, Wn0):
    def body(f_ref, wn_ref, y0_ref):
        y0_ref[...] = _split(_bdot(f_ref[...], wn_ref[...]))

    return pl.pallas_call(
        body,
        grid=(GRID,),
        in_specs=[_rowspec(D), _fullspec((D, D))],
        out_specs=_stkspec(),
        out_shape=jax.ShapeDtypeStruct((NC, N_PAD, DH), _f32),
    )(feat, Wn0)


def _tc_a2(feat, gfeat, Ws0, b0, Wpb, bp):
    def body(f_ref, g_ref, ws_ref, b0_ref, wpb_ref, bp_ref, s0_ref, gp_ref):
        s0_ref[...] = _bdot(f_ref[...], ws_ref[...]) + b0_ref[...]
        gp_ref[...] = _bdot(g_ref[...], wpb_ref[...]) + bp_ref[...]

    return pl.pallas_call(
        body,
        grid=(GRID,),
        in_specs=[_rowspec(D), _rowspec(D), _fullspec((D, D)),
                  _fullspec((1, D)), _fullspec((D, C)), _fullspec((1, C))],
        out_specs=[_rowspec(D), _rowspec(C)],
        out_shape=[jax.ShapeDtypeStruct((N_PAD, D), _f32),
                   jax.ShapeDtypeStruct((N_PAD, C), _f32)],
    )(feat, gfeat, Ws0, b0, Wpb, bp)


def _h1_of(sum_ref, deg_ref, s0_ref):
    d = _deg_of(deg_ref)
    agg = jnp.concatenate([sum_ref[0], sum_ref[1]], axis=-1).astype(_f32)
    return jnp.maximum(s0_ref[...] + agg / d, 0.0)


def _tc_b1(sum0, deg, s0, Wn1):
    def body(sum_ref, deg_ref, s0_ref, wn_ref, y1_ref):
        h1 = _h1_of(sum_ref, deg_ref, s0_ref)
        y1_ref[...] = _split(_bdot(h1, wn_ref[...]))

    return pl.pallas_call(
        body,
        grid=(GRID,),
        in_specs=[_stkspec(), _degspec(), _rowspec(D), _fullspec((D, D))],
        out_specs=_stkspec(),
        out_shape=jax.ShapeDtypeStruct((NC, N_PAD, DH), _f32),
    )(sum0, deg, s0, Wn1)


def _tc_b2(sum0, deg, s0, Ws1, b1):
    def body(sum_ref, deg_ref, s0_ref, ws_ref, b1_ref, s1_ref):
        h1 = _h1_of(sum_ref, deg_ref, s0_ref)
        s1_ref[...] = _bdot(h1, ws_ref[...]) + b1_ref[...]

    return pl.pallas_call(
        body,
        grid=(GRID,),
        in_specs=[_stkspec(), _degspec(), _rowspec(D), _fullspec((D, D)),
                  _fullspec((1, D))],
        out_specs=_rowspec(D),
        out_shape=jax.ShapeDtypeStruct((N_PAD, D), _f32),
    )(sum0, deg, s0, Ws1, b1)


def _tc_c(sum1, deg, s1, gp, Wpt):
    def body(sum_ref, deg_ref, s1_ref, gp_ref, wpt_ref, out_ref):
        h2 = _h1_of(sum_ref, deg_ref, s1_ref)
        out_ref[...] = _bdot(h2, wpt_ref[...]) + gp_ref[...]

    return pl.pallas_call(
        body,
        grid=(GRID,),
        in_specs=[_stkspec(), _degspec(), _rowspec(D), _rowspec(C),
                  _fullspec((D, C))],
        out_specs=_rowspec(C),
        out_shape=jax.ShapeDtypeStruct((N_PAD, C), _f32),
    )(sum1, deg, s1, gp, Wpt)


def kernel(feat, gfeat, edge_index, W_self0, W_neigh0, b0,
           W_self1, W_neigh1, b1, W_p, b_p):
    feat = jnp.pad(feat, ((0, N_PAD - N), (0, 0)))
    gfeat = jnp.pad(gfeat, ((0, N_PAD - N), (0, 0)))
    src = edge_index[0].astype(jnp.int32)
    dst = edge_index[1].astype(jnp.int32)
    pad = jnp.arange(E_PAD - E, dtype=jnp.int32)
    src = jnp.concatenate([src, pad % N])
    dst = jnp.concatenate([dst, N + pad % (N_PAD - N)])
    srcr = src.reshape(E_PAD // K, K)
    dstr = dst.reshape(E_PAD // K, K)
    idx_pack = jnp.concatenate([
        jnp.stack([srcr, dstr], axis=1),
        jnp.stack([srcr + N_PAD, dstr], axis=1),
    ]).reshape(NC * (E_PAD // K) * 2, K)

    b0r = b0.reshape(1, D)
    b1r = b1.reshape(1, D)
    bpr = b_p.reshape(1, C)
    Wpt = W_p[:D]
    Wpb = W_p[D:]

    y0 = _tc_a1(feat, W_neigh0)
    sum0, deg = _sc_agg(True)(y0.reshape(NC * N_PAD, DH), idx_pack)
    s0, gp = _tc_a2(feat, gfeat, W_self0, b0r, Wpb, bpr)
    degc = deg.reshape(NC, N_PAD, 1)
    sum0s = sum0.reshape(NC, N_PAD, DH)
    y1 = _tc_b1(sum0s, degc, s0, W_neigh1)
    sum1, _ = _sc_agg(False)(y1.reshape(NC * N_PAD, DH), idx_pack)
    s1 = _tc_b2(sum0s, degc, s0, W_self1, b1r)
    out = _tc_c(sum1.reshape(NC, N_PAD, DH), degc, s1, gp, Wpt)
    return out[:N]


# speedup vs baseline: 1.0216x; 1.0201x over previous
import functools

import jax
import jax.numpy as jnp
from jax import lax
from jax.experimental import pallas as pl
from jax.experimental.pallas import tpu as pltpu
from jax.experimental.pallas import tpu_sc as plsc

N = 10000
E = 160000
D = 256
C = 64

NC = 2
NS = 16
L = 16
DH = D // NC

N_PAD = 10240
K = 128
E_PAD = 163840
CH_A = E_PAD // (NS * K)
ZROWS = N_PAD // NS


def _mesh():
    return plsc.VectorSubcoreMesh(core_axis_name="c", subcore_axis_name="s",
                                  num_cores=NC, num_subcores=NS)


NBUF = 2
ZB = 32


@functools.lru_cache(maxsize=None)
def _sc_agg(with_deg):
    scratch = (
        [pltpu.VMEM((K, DH), jnp.float32) for _ in range(NBUF)] +
        [pltpu.VMEM((2, K), jnp.int32) for _ in range(NBUF)] +
        [
            pltpu.VMEM((ZB, DH), jnp.float32),
            pltpu.VMEM((K,), jnp.float32),
            pltpu.VMEM((ZROWS,), jnp.float32),
            pltpu.VMEM_SHARED((N_PAD, DH), jnp.float32),
            pltpu.VMEM_SHARED((N_PAD,), jnp.float32),
        ] +
        [pltpu.SemaphoreType.DMA for _ in range(NBUF)]
    )

    def body(y_hbm, idx_hbm, out_hbm, deg_hbm, *sc):
        rows = list(sc[:NBUF])
        idx = list(sc[NBUF:2 * NBUF])
        zrow_v, ones_v, zdeg_v, acc_s, dacc_s = sc[2 * NBUF:2 * NBUF + 5]
        sems = list(sc[2 * NBUF + 5:])
        c = lax.axis_index("c")
        t = lax.axis_index("s")
        zeros16 = jnp.zeros((L,), jnp.float32)
        ones16 = jnp.ones((L,), jnp.float32)

        for i in range(ZB):
            for j in range(DH // L):
                zrow_v[i, pl.ds(j * L, L)] = zeros16
        for i in range(K // L):
            ones_v[pl.ds(i * L, L)] = ones16
        for i in range(ZROWS // L):
            zdeg_v[pl.ds(i * L, L)] = zeros16

        zbase = t * ZROWS
        for i in range(ZROWS // ZB):
            pltpu.async_copy(zrow_v, acc_s.at[pl.ds(zbase + i * ZB, ZB), :],
                             sems[0])
        pltpu.sync_copy(zdeg_v, dacc_s.at[pl.ds(zbase, ZROWS)])
        for i in range(ZROWS // ZB):
            pltpu.make_async_copy(zrow_v, acc_s.at[pl.ds(zbase + i * ZB, ZB), :],
                                  sems[0]).wait()

        rbase0 = 2 * (c * (E_PAD // K) + t * CH_A)

        def fire(ci, b):
            pltpu.sync_copy(idx_hbm.at[pl.ds(rbase0 + 2 * ci, 2)], idx[b])
            pltpu.async_copy(y_hbm.at[idx[b].at[0]], rows[b], sems[b])

        for b in range(NBUF - 1):
            fire(b, b)
        plsc.subcore_barrier()

        def group_body(g, _):
            for b in range(NBUF):
                ci = g * NBUF + b
                cf = ci + NBUF - 1

                @pl.when(cf < CH_A)
                def _():
                    fire(cf, (b + NBUF - 1) % NBUF)

                pltpu.make_async_copy(y_hbm.at[idx[b].at[0]],
                                      rows[b], sems[b]).wait()
                pltpu.sync_copy(rows[b], acc_s.at[idx[b].at[1]], add=True)

                if with_deg:
                    @pl.when(lax.rem(ci, 2) == c)
                    def _():
                        pltpu.sync_copy(ones_v, dacc_s.at[idx[b].at[1]],
                                        add=True)
            return ()

        lax.fori_loop(0, CH_A // NBUF, group_body, (), unroll=False)
        plsc.subcore_barrier()

        pltpu.sync_copy(acc_s.at[pl.ds(zbase, ZROWS)],
                        out_hbm.at[pl.ds(c * N_PAD + zbase, ZROWS)])

        if with_deg:
            pltpu.sync_copy(dacc_s.at[pl.ds(zbase, ZROWS)],
                            deg_hbm.at[pl.ds(c * N_PAD + zbase, ZROWS)])

    return pl.kernel(
        body,
        out_type=[jax.ShapeDtypeStruct((NC * N_PAD, DH), jnp.float32),
                  jax.ShapeDtypeStruct((NC * N_PAD,), jnp.float32)],
        mesh=_mesh(), scratch_types=scratch)


RB = 1024
GRID = N_PAD // RB

_f32 = jnp.float32
_bf = jnp.bfloat16


def _bdot(a, b):
    return jnp.dot(a, b, preferred_element_type=_f32)


def _rowspec(w):
    return pl.BlockSpec((RB, w), lambda i: (i, 0))


def _stkspec():
    return pl.BlockSpec((NC, RB, DH), lambda i: (0, i, 0))


def _fullspec(shape):
    return pl.BlockSpec(shape, lambda i: tuple(0 for _ in shape))


def _split(y):
    return jnp.stack([y[:, :DH], y[:, DH:]], axis=0)


def _degspec():
    return pl.BlockSpec((NC, RB, 1), lambda i: (0, i, 0))


def _deg_of(deg_ref):
    return jnp.maximum(deg_ref[0] + deg_ref[1], 1.0)


def _tc_a1(feat, Wn0):
    def body(f_ref, wn_ref, y0_ref):
        y0_ref[...] = _split(_bdot(f_ref[...], wn_ref[...]))

    return pl.pallas_call(
        body,
        grid=(GRID,),
        in_specs=[_rowspec(D), _fullspec((D, D))],
        out_specs=_stkspec(),
        out_shape=jax.ShapeDtypeStruct((NC, N_PAD, DH), _f32),
    )(feat, Wn0)


def _tc_a2(feat, gfeat, Ws0, b0, Wpb, bp):
    def body(f_ref, g_ref, ws_ref, b0_ref, wpb_ref, bp_ref, s0_ref, gp_ref):
        s0_ref[...] = (_bdot(f_ref[...], ws_ref[...]) + b0_ref[...]).astype(_bf)
        gp_ref[...] = (_bdot(g_ref[...], wpb_ref[...]) + bp_ref[...]).astype(_bf)

    return pl.pallas_call(
        body,
        grid=(GRID,),
        in_specs=[_rowspec(D), _rowspec(D), _fullspec((D, D)),
                  _fullspec((1, D)), _fullspec((D, C)), _fullspec((1, C))],
        out_specs=[_rowspec(D), _rowspec(C)],
        out_shape=[jax.ShapeDtypeStruct((N_PAD, D), _bf),
                   jax.ShapeDtypeStruct((N_PAD, C), _bf)],
    )(feat, gfeat, Ws0, b0, Wpb, bp)


def _h1_of(sum_ref, deg_ref, s0_ref):
    d = _deg_of(deg_ref)
    agg = jnp.concatenate([sum_ref[0], sum_ref[1]], axis=-1).astype(_f32)
    return jnp.maximum(s0_ref[...].astype(_f32) + agg / d, 0.0)


def _tc_b1(sum0, deg, s0, Wn1):
    def body(sum_ref, deg_ref, s0_ref, wn_ref, y1_ref):
        h1 = _h1_of(sum_ref, deg_ref, s0_ref)
        y1_ref[...] = _split(_bdot(h1, wn_ref[...]))

    return pl.pallas_call(
        body,
        grid=(GRID,),
        in_specs=[_stkspec(), _degspec(), _rowspec(D), _fullspec((D, D))],
        out_specs=_stkspec(),
        out_shape=jax.ShapeDtypeStruct((NC, N_PAD, DH), _f32),
    )(sum0, deg, s0, Wn1)


def _tc_b2(sum0, deg, s0, Ws1, b1):
    def body(sum_ref, deg_ref, s0_ref, ws_ref, b1_ref, s1_ref):
        h1 = _h1_of(sum_ref, deg_ref, s0_ref)
        s1_ref[...] = (_bdot(h1, ws_ref[...]) + b1_ref[...]).astype(_bf)

    return pl.pallas_call(
        body,
        grid=(GRID,),
        in_specs=[_stkspec(), _degspec(), _rowspec(D), _fullspec((D, D)),
                  _fullspec((1, D))],
        out_specs=_rowspec(D),
        out_shape=jax.ShapeDtypeStruct((N_PAD, D), _bf),
    )(sum0, deg, s0, Ws1, b1)


def _tc_c(sum1, deg, s1, gp, Wpt):
    def body(sum_ref, deg_ref, s1_ref, gp_ref, wpt_ref, out_ref):
        h2 = _h1_of(sum_ref, deg_ref, s1_ref)
        out_ref[...] = _bdot(h2, wpt_ref[...]) + gp_ref[...].astype(_f32)

    return pl.pallas_call(
        body,
        grid=(GRID,),
        in_specs=[_stkspec(), _degspec(), _rowspec(D), _rowspec(C),
                  _fullspec((D, C))],
        out_specs=_rowspec(C),
        out_shape=jax.ShapeDtypeStruct((N_PAD, C), _f32),
    )(sum1, deg, s1, gp, Wpt)


def kernel(feat, gfeat, edge_index, W_self0, W_neigh0, b0,
           W_self1, W_neigh1, b1, W_p, b_p):
    feat = jnp.pad(feat, ((0, N_PAD - N), (0, 0)))
    gfeat = jnp.pad(gfeat, ((0, N_PAD - N), (0, 0)))
    src = edge_index[0].astype(jnp.int32)
    dst = edge_index[1].astype(jnp.int32)
    pad = jnp.arange(E_PAD - E, dtype=jnp.int32)
    src = jnp.concatenate([src, pad % N])
    dst = jnp.concatenate([dst, N + pad % (N_PAD - N)])
    srcr = src.reshape(E_PAD // K, K)
    dstr = dst.reshape(E_PAD // K, K)
    idx_pack = jnp.concatenate([
        jnp.stack([srcr, dstr], axis=1),
        jnp.stack([srcr + N_PAD, dstr], axis=1),
    ]).reshape(NC * (E_PAD // K) * 2, K)

    b0r = b0.reshape(1, D)
    b1r = b1.reshape(1, D)
    bpr = b_p.reshape(1, C)
    Wpt = W_p[:D]
    Wpb = W_p[D:]

    y0 = _tc_a1(feat, W_neigh0)
    sum0, deg = _sc_agg(True)(y0.reshape(NC * N_PAD, DH), idx_pack)
    s0, gp = _tc_a2(feat, gfeat, W_self0, b0r, Wpb, bpr)
    degc = deg.reshape(NC, N_PAD, 1)
    sum0s = sum0.reshape(NC, N_PAD, DH)
    y1 = _tc_b1(sum0s, degc, s0, W_neigh1)
    sum1, _ = _sc_agg(False)(y1.reshape(NC * N_PAD, DH), idx_pack)
    s1 = _tc_b2(sum0s, degc, s0, W_self1, b1r)
    out = _tc_c(sum1.reshape(NC, N_PAD, DH), degc, s1, gp, Wpt)
    return out[:N]
